# Initial kernel scaffold; baseline (speedup 1.0000x reference)
import jax
import jax.numpy as jnp
from jax.experimental import pallas as pl


def kernel(img, U):
    raise NotImplementedError("write your pallas kernel here")


# speedup vs baseline: 1.6169x; 1.6169x over previous
import functools

import jax
import jax.numpy as jnp
from jax import lax
from jax.experimental import pallas as pl
from jax.experimental.pallas import tpu as pltpu
from jax.experimental.pallas import tpu_sc as plsc

B, F, H, W = 4, 8, 384, 384
NC, NS, L = 2, 16, 16
NW = NC * NS
ROWS_PER_CHUNK = 16
NCHUNK = H // ROWS_PER_CHUNK
UCHUNK = ROWS_PER_CHUNK * W * 2
OCHUNK = ROWS_PER_CHUNK * W
NSTEP = W // L
TABLE_WORDS = H * W // 2


def _warp_body(packed_hbm, u_hbm, out_hbm, table_v, u_v, out_v):
    c = lax.axis_index("c")
    s = lax.axis_index("s")
    wid = s * NC + c
    b = wid // F
    f = wid % F

    pltpu.sync_copy(packed_hbm.at[b], table_v)

    lane = lax.iota(jnp.int32, L)
    lane_f = lane.astype(jnp.float32)

    def chunk_body(ck, _):
        pltpu.sync_copy(u_hbm.at[b, f, ck], u_v)

        def row_body(r, _):
            row_f = lax.broadcast_in_dim(
                (ck * ROWS_PER_CHUNK + r).astype(jnp.float32), (L,), ())
            ubase = r * (W * 2)
            obase = r * W
            for st in range(NSTEP):
                idx_d = ubase + (st * 2 * L) + 2 * lane
                di = plsc.load_gather(u_v, [idx_d])
                dj = plsc.load_gather(u_v, [idx_d + 1])
                si = row_f - di
                sj = (jnp.float32(st * L) + lane_f) - dj
                i0 = jnp.clip(si.astype(jnp.int32), 0, H - 1)
                j0 = jnp.clip(sj.astype(jnp.int32), 0, W - 1)
                i1 = jnp.minimum(i0 + 1, H - 1)
                j1 = jnp.minimum(j0 + 1, W - 1)
                dif = si - i0.astype(jnp.float32)
                djf = sj - j0.astype(jnp.float32)
                q00 = i0 * W + j0
                dj1 = j1 - j0
                q01 = q00 + dj1
                q10 = i1 * W + j0
                q11 = q10 + dj1

                def tap(q):
                    w = plsc.load_gather(table_v, [lax.shift_right_logical(q, 1)])
                    lo = lax.shift_left(w, 16)
                    hi = jnp.bitwise_and(w, jnp.int32(-65536))
                    bits = jnp.where(jnp.bitwise_and(q, 1) == 1, hi, lo)
                    return plsc.bitcast(bits, jnp.float32)

                f00 = tap(q00)
                f01 = tap(q01)
                f10 = tap(q10)
                f11 = tap(q11)
                f0 = f00 + (f01 - f00) * djf
                f1 = f10 + (f11 - f10) * djf
                out_v[pl.ds(obase + st * L, L)] = f0 + (f1 - f0) * dif
            return 0

        lax.fori_loop(0, ROWS_PER_CHUNK, row_body, 0)
        pltpu.sync_copy(out_v, out_hbm.at[wid, ck])
        return 0

    lax.fori_loop(0, NCHUNK, chunk_body, 0)


@functools.partial(
    pl.kernel,
    out_type=jax.ShapeDtypeStruct((NW, NCHUNK, OCHUNK), jnp.float32),
    mesh=plsc.VectorSubcoreMesh(core_axis_name="c", subcore_axis_name="s"),
    compiler_params=pltpu.CompilerParams(needs_layout_passes=False),
    scratch_types=[
        pltpu.VMEM((TABLE_WORDS,), jnp.int32),
        pltpu.VMEM((UCHUNK,), jnp.float32),
        pltpu.VMEM((OCHUNK,), jnp.float32),
    ],
)
def _warp_kernel(packed_hbm, u_hbm, out_hbm, table_v, u_v, out_v):
    _warp_body(packed_hbm, u_hbm, out_hbm, table_v, u_v, out_v)


def kernel(img, U):
    packed = lax.bitcast_convert_type(
        img.astype(jnp.bfloat16).reshape(B, TABLE_WORDS, 2), jnp.int32)
    u = U.reshape(B, F, NCHUNK, UCHUNK)
    out = _warp_kernel(packed, u)
    return out.reshape(B, F, H, W)


# speedup vs baseline: 2.3169x; 1.4329x over previous
import functools

import jax
import jax.numpy as jnp
from jax import lax
from jax.experimental import pallas as pl
from jax.experimental.pallas import tpu as pltpu
from jax.experimental.pallas import tpu_sc as plsc

B, F, H, W = 4, 8, 384, 384
NC, NS, L = 2, 16, 16
NW = NC * NS
WT, WC = W // 128, 128
ROWS_PER_CHUNK = 16
NCHUNK = H // ROWS_PER_CHUNK
NSTEP = W // L
TABLE_WORDS = H * W // 2


def _warp_body(packed_hbm, u_hbm, out_hbm, table_v, u_v, out_v):
    c = lax.axis_index("c")
    s = lax.axis_index("s")
    wid = s * NC + c
    b = wid // F
    f = wid % F

    pltpu.sync_copy(packed_hbm.at[b], table_v)

    lane = lax.iota(jnp.int32, L)
    lane_f = lane.astype(jnp.float32)

    def chunk_body(ck, _):
        row0 = ck * ROWS_PER_CHUNK
        pltpu.sync_copy(u_hbm.at[b, f, pl.ds(row0, ROWS_PER_CHUNK)], u_v)

        def row_body(r, _):
            row_f = lax.broadcast_in_dim(
                (row0 + r).astype(jnp.float32), (L,), ())
            for st in range(NSTEP):
                t, woff = (st * L) // WC, (st * L) % WC
                di = u_v[r, t, 0, pl.ds(woff, L)]
                dj = u_v[r, t, 1, pl.ds(woff, L)]
                si = row_f - di
                sj = (jnp.float32(st * L) + lane_f) - dj
                i0 = jnp.clip(si.astype(jnp.int32), 0, H - 1)
                j0 = jnp.clip(sj.astype(jnp.int32), 0, W - 1)
                i1 = jnp.minimum(i0 + 1, H - 1)
                j1 = jnp.minimum(j0 + 1, W - 1)
                dif = si - i0.astype(jnp.float32)
                djf = sj - j0.astype(jnp.float32)
                q00 = i0 * W + j0
                dj1 = j1 - j0
                q01 = q00 + dj1
                q10 = i1 * W + j0
                q11 = q10 + dj1

                def tap(q):
                    w = plsc.load_gather(table_v, [lax.shift_right_logical(q, 1)])
                    lo = lax.shift_left(w, 16)
                    hi = jnp.bitwise_and(w, jnp.int32(-65536))
                    bits = jnp.where(jnp.bitwise_and(q, 1) == 1, hi, lo)
                    return plsc.bitcast(bits, jnp.float32)

                f00 = tap(q00)
                f01 = tap(q01)
                f10 = tap(q10)
                f11 = tap(q11)
                f0 = f00 + (f01 - f00) * djf
                f1 = f10 + (f11 - f10) * djf
                out_v[r, pl.ds(st * L, L)] = f0 + (f1 - f0) * dif
            return 0

        lax.fori_loop(0, ROWS_PER_CHUNK, row_body, 0)
        pltpu.sync_copy(out_v, out_hbm.at[b, f, pl.ds(row0, ROWS_PER_CHUNK)])
        return 0

    lax.fori_loop(0, NCHUNK, chunk_body, 0)


@functools.partial(
    pl.kernel,
    out_type=jax.ShapeDtypeStruct((B, F, H, W), jnp.float32),
    mesh=plsc.VectorSubcoreMesh(core_axis_name="c", subcore_axis_name="s"),
    compiler_params=pltpu.CompilerParams(needs_layout_passes=False),
    scratch_types=[
        pltpu.VMEM((TABLE_WORDS,), jnp.int32),
        pltpu.VMEM((ROWS_PER_CHUNK, WT, 2, WC), jnp.float32),
        pltpu.VMEM((ROWS_PER_CHUNK, W), jnp.float32),
    ],
)
def _warp_kernel(packed_hbm, u_hbm, out_hbm, table_v, u_v, out_v):
    _warp_body(packed_hbm, u_hbm, out_hbm, table_v, u_v, out_v)


def kernel(img, U):
    packed = lax.bitcast_convert_type(
        img.astype(jnp.bfloat16).reshape(B, TABLE_WORDS, 2), jnp.int32)
    u = U.reshape(B, F, H, WT, WC, 2).transpose(0, 1, 2, 3, 5, 4)
    return _warp_kernel(packed, u)


# speedup vs baseline: 5.2116x; 2.2493x over previous
import functools

import jax
import jax.numpy as jnp
from jax import lax
from jax.experimental import pallas as pl
from jax.experimental.pallas import tpu as pltpu
from jax.experimental.pallas import tpu_sc as plsc

B, F, H, W = 4, 8, 384, 384
NC, NS, L = 2, 16, 16
NW = NC * NS
WT, WC = W // 128, 128
ROWS_PER_CHUNK = 16
NCHUNK = H // ROWS_PER_CHUNK
NSTEP = W // L
PACK_ROWS = 32
NPCHUNK = H // PACK_ROWS
TABLE_WORDS = H * W // 2


def _warp_body(img_hbm, u_hbm, out_hbm, table_v, u_v, out_v, stage_v):
    c = lax.axis_index("c")
    s = lax.axis_index("s")
    wid = s * NC + c
    b = wid // F
    f = wid % F

    lane = lax.iota(jnp.int32, L)
    lane_f = lane.astype(jnp.float32)

    def pack_chunk(pk, _):
        pltpu.sync_copy(img_hbm.at[b, pl.ds(pk * PACK_ROWS, PACK_ROWS)], stage_v)

        def pack_row(rp, _):
            wbase = (pk * (PACK_ROWS // 2) + rp) * W
            for st in range(NSTEP):
                va = stage_v[2 * rp, pl.ds(st * L, L)]
                vb = stage_v[2 * rp + 1, pl.ds(st * L, L)]
                packed = plsc.pack(va, vb, format=plsc.PackFormat.INTERLEAVED)
                table_v[pl.ds(wbase + st * L, L)] = plsc.bitcast(packed, jnp.int32)
            return 0

        lax.fori_loop(0, PACK_ROWS // 2, pack_row, 0)
        return 0

    lax.fori_loop(0, NPCHUNK, pack_chunk, 0)

    def chunk_body(ck, _):
        row0 = ck * ROWS_PER_CHUNK
        pltpu.sync_copy(u_hbm.at[b, f, pl.ds(row0, ROWS_PER_CHUNK)], u_v)

        def row_body(r, _):
            row_f = lax.broadcast_in_dim(
                (row0 + r).astype(jnp.float32), (L,), ())
            for st in range(NSTEP):
                t, woff = (st * L) // WC, (st * L) % WC
                di = u_v[r, t, 0, pl.ds(woff, L)]
                dj = u_v[r, t, 1, pl.ds(woff, L)]
                si = row_f - di
                sj = (jnp.float32(st * L) + lane_f) - dj
                i0 = jnp.clip(si.astype(jnp.int32), 0, H - 1)
                j0 = jnp.clip(sj.astype(jnp.int32), 0, W - 1)
                i1 = jnp.minimum(i0 + 1, H - 1)
                j1 = jnp.minimum(j0 + 1, W - 1)
                dif = si - i0.astype(jnp.float32)
                djf = sj - j0.astype(jnp.float32)
                qw0 = lax.shift_right_logical(i0, 1) * W + j0
                qw1 = lax.shift_right_logical(i1, 1) * W + j0
                dj1 = j1 - j0
                p0 = lax.shift_left(jnp.bitwise_and(i0, 1), 4)
                p1 = lax.shift_left(jnp.bitwise_and(i1, 1), 4)

                def tap(qw, psh):
                    w = plsc.load_gather(table_v, [qw])
                    bits = lax.shift_left(w, 16 - psh)
                    bits = jnp.bitwise_and(bits, jnp.int32(-65536))
                    return plsc.bitcast(bits, jnp.float32)

                f00 = tap(qw0, p0)
                f01 = tap(qw0 + dj1, p0)
                f10 = tap(qw1, p1)
                f11 = tap(qw1 + dj1, p1)
                f0 = f00 + (f01 - f00) * djf
                f1 = f10 + (f11 - f10) * djf
                out_v[r, pl.ds(st * L, L)] = f0 + (f1 - f0) * dif
            return 0

        lax.fori_loop(0, ROWS_PER_CHUNK, row_body, 0)
        pltpu.sync_copy(out_v, out_hbm.at[b, f, pl.ds(row0, ROWS_PER_CHUNK)])
        return 0

    lax.fori_loop(0, NCHUNK, chunk_body, 0)


@functools.partial(
    pl.kernel,
    out_type=jax.ShapeDtypeStruct((B, F, H, W), jnp.float32),
    mesh=plsc.VectorSubcoreMesh(core_axis_name="c", subcore_axis_name="s"),
    compiler_params=pltpu.CompilerParams(needs_layout_passes=False),
    scratch_types=[
        pltpu.VMEM((TABLE_WORDS,), jnp.int32),
        pltpu.VMEM((ROWS_PER_CHUNK, WT, 2, WC), jnp.float32),
        pltpu.VMEM((ROWS_PER_CHUNK, W), jnp.float32),
        pltpu.VMEM((PACK_ROWS, W), jnp.float32),
    ],
)
def _warp_kernel(img_hbm, u_hbm, out_hbm, table_v, u_v, out_v, stage_v):
    _warp_body(img_hbm, u_hbm, out_hbm, table_v, u_v, out_v, stage_v)


def kernel(img, U):
    u = U.reshape(B, F, H, WT, WC, 2).transpose(0, 1, 2, 3, 5, 4)
    return _warp_kernel(img, u)


# speedup vs baseline: 9.1709x; 1.7597x over previous
import functools

import jax
import jax.numpy as jnp
from jax import lax
from jax.experimental import pallas as pl
from jax.experimental.pallas import tpu as pltpu
from jax.experimental.pallas import tpu_sc as plsc

B, F, H, W = 4, 8, 384, 384
NC, NS, L = 2, 16, 16
NW = NC * NS
WT, WC = W // 128, 128
ROWS_PER_CHUNK = 16
NCHUNK = H // ROWS_PER_CHUNK
NSTEP = W // L
PACK_ROWS = 32
NPCHUNK = H // PACK_ROWS
TABLE_WORDS = H * W // 2


def _warp_body(img_hbm, u_hbm, out_hbm, table_v, u_v, out_v, stage_v):
    c = lax.axis_index("c")
    s = lax.axis_index("s")
    wid = s * NC + c
    b = wid // F
    f = wid % F

    lane = lax.iota(jnp.int32, L)
    lane_f = lane.astype(jnp.float32)

    def pack_chunk(pk, _):
        pltpu.sync_copy(img_hbm.at[b, pl.ds(pk * PACK_ROWS, PACK_ROWS)], stage_v)

        @plsc.parallel_loop(0, PACK_ROWS // 2)
        def pack_row(rp):
            wbase = (pk * (PACK_ROWS // 2) + rp) * W
            for st in range(NSTEP):
                va = stage_v[2 * rp + 1, pl.ds(st * L, L)]
                vb = stage_v[2 * rp, pl.ds(st * L, L)]
                packed = plsc.pack(va, vb, format=plsc.PackFormat.INTERLEAVED)
                table_v[pl.ds(wbase + st * L, L)] = plsc.bitcast(packed, jnp.int32)

        return 0

    lax.fori_loop(0, NPCHUNK, pack_chunk, 0)

    def chunk_body(ck, _):
        row0 = ck * ROWS_PER_CHUNK
        pltpu.sync_copy(u_hbm.at[b, f, pl.ds(row0, ROWS_PER_CHUNK)], u_v)

        @plsc.parallel_loop(0, ROWS_PER_CHUNK)
        def row_body(r):
            row_f = lax.broadcast_in_dim(
                (row0 + r).astype(jnp.float32), (L,), ())

            def umin(x, k):
                return plsc.bitcast(
                    jnp.minimum(plsc.bitcast(x, jnp.uint32), jnp.uint32(k)),
                    jnp.int32)

            GROUP = 4
            for st0 in range(0, NSTEP, GROUP):
                sts = range(st0, st0 + GROUP)
                di = [u_v[r, (st * L) // WC, 0, pl.ds((st * L) % WC, L)]
                      for st in sts]
                dj = [u_v[r, (st * L) // WC, 1, pl.ds((st * L) % WC, L)]
                      for st in sts]
                si = [row_f - d for d in di]
                sj = [(jnp.float32(st * L) + lane_f) - d
                      for st, d in zip(sts, dj)]
                i0 = [umin(jnp.maximum(x, 0.0).astype(jnp.int32), H - 1)
                      for x in si]
                j0 = [umin(jnp.maximum(x, 0.0).astype(jnp.int32), W - 1)
                      for x in sj]
                i1 = [umin(x + 1, H - 1) for x in i0]
                j1 = [umin(x + 1, W - 1) for x in j0]
                dif = [x - v.astype(jnp.float32) for x, v in zip(si, i0)]
                djf = [x - v.astype(jnp.float32) for x, v in zip(sj, j0)]
                qw0 = [lax.shift_right_logical(a, 1) * W + b
                       for a, b in zip(i0, j0)]
                qw1 = [lax.shift_right_logical(a, 1) * W + b
                       for a, b in zip(i1, j0)]
                dj1 = [a - b for a, b in zip(j1, j0)]
                p0 = [lax.shift_left(jnp.bitwise_and(a, 1), 4) for a in i0]
                p1 = [lax.shift_left(jnp.bitwise_and(a, 1), 4) for a in i1]

                def tap(qw, psh):
                    w = plsc.load_gather(table_v, [qw])
                    bits = lax.shift_left(w, psh)
                    bits = jnp.bitwise_and(bits, jnp.int32(-65536))
                    return plsc.bitcast(bits, jnp.float32)

                f00 = [tap(q, p) for q, p in zip(qw0, p0)]
                f01 = [tap(q + d, p) for q, d, p in zip(qw0, dj1, p0)]
                f10 = [tap(q, p) for q, p in zip(qw1, p1)]
                f11 = [tap(q + d, p) for q, d, p in zip(qw1, dj1, p1)]
                f0 = [a + (b - a) * w for a, b, w in zip(f00, f01, djf)]
                f1 = [a + (b - a) * w for a, b, w in zip(f10, f11, djf)]
                for k, st in enumerate(sts):
                    out_v[r, pl.ds(st * L, L)] = (
                        f0[k] + (f1[k] - f0[k]) * dif[k])
        pltpu.sync_copy(out_v, out_hbm.at[b, f, pl.ds(row0, ROWS_PER_CHUNK)])
        return 0

    lax.fori_loop(0, NCHUNK, chunk_body, 0)


@functools.partial(
    pl.kernel,
    out_type=jax.ShapeDtypeStruct((B, F, H, W), jnp.float32),
    mesh=plsc.VectorSubcoreMesh(core_axis_name="c", subcore_axis_name="s"),
    compiler_params=pltpu.CompilerParams(needs_layout_passes=False),
    scratch_types=[
        pltpu.VMEM((TABLE_WORDS,), jnp.int32),
        pltpu.VMEM((ROWS_PER_CHUNK, WT, 2, WC), jnp.float32),
        pltpu.VMEM((ROWS_PER_CHUNK, W), jnp.float32),
        pltpu.VMEM((PACK_ROWS, W), jnp.float32),
    ],
)
def _warp_kernel(img_hbm, u_hbm, out_hbm, table_v, u_v, out_v, stage_v):
    _warp_body(img_hbm, u_hbm, out_hbm, table_v, u_v, out_v, stage_v)


def kernel(img, U):
    u = U.reshape(B, F, H, WT, WC, 2).transpose(0, 1, 2, 3, 5, 4)
    return _warp_kernel(img, u)


# speedup vs baseline: 9.5104x; 1.0370x over previous
import functools

import jax
import jax.numpy as jnp
from jax import lax
from jax.experimental import pallas as pl
from jax.experimental.pallas import tpu as pltpu
from jax.experimental.pallas import tpu_sc as plsc

B, F, H, W = 4, 8, 384, 384
NC, NS, L = 2, 16, 16
NW = NC * NS
WT, WC = W // 128, 128
ROWS_PER_CHUNK = 16
NCHUNK = H // ROWS_PER_CHUNK
NSTEP = W // L
PACK_ROWS = 32
NPCHUNK = H // PACK_ROWS
TABLE_WORDS = H * W // 2


def _warp_body(img_hbm, u_hbm, out_hbm, table_v, u_v, out_v, stage_v):
    c = lax.axis_index("c")
    s = lax.axis_index("s")
    wid = s * NC + c
    b = wid // F
    f = wid % F

    lane = lax.iota(jnp.int32, L)
    lane_f = lane.astype(jnp.float32)

    def pack_chunk(pk, _):
        pltpu.sync_copy(img_hbm.at[b, pl.ds(pk * PACK_ROWS, PACK_ROWS)], stage_v)

        @plsc.parallel_loop(0, PACK_ROWS // 2)
        def pack_row(rp):
            wrow = pk * (PACK_ROWS // 2) + rp
            for st in range(NSTEP):
                va = stage_v[2 * rp + 1, pl.ds(st * L, L)]
                vb = stage_v[2 * rp, pl.ds(st * L, L)]
                packed = plsc.pack(va, vb, format=plsc.PackFormat.INTERLEAVED)
                table_v[pl.ds(wrow * W + st * L, L)] = plsc.bitcast(
                    packed, jnp.int32)

        return 0

    lax.fori_loop(0, NPCHUNK, pack_chunk, 0)

    def chunk_body(ck, _):
        row0 = ck * ROWS_PER_CHUNK
        pltpu.sync_copy(u_hbm.at[b, f, pl.ds(row0, ROWS_PER_CHUNK)], u_v)

        @plsc.parallel_loop(0, ROWS_PER_CHUNK)
        def row_body(r):
            row_f = lax.broadcast_in_dim(
                (row0 + r).astype(jnp.float32), (L,), ())

            def umin(x, k):
                return plsc.bitcast(
                    jnp.minimum(plsc.bitcast(x, jnp.uint32), jnp.uint32(k)),
                    jnp.int32)

            GROUP = 4
            for st0 in range(0, NSTEP, GROUP):
                sts = range(st0, st0 + GROUP)
                di = [u_v[r, (st * L) // WC, 0, pl.ds((st * L) % WC, L)]
                      for st in sts]
                dj = [u_v[r, (st * L) // WC, 1, pl.ds((st * L) % WC, L)]
                      for st in sts]
                si = [row_f - d for d in di]
                sj = [(jnp.float32(st * L) + lane_f) - d
                      for st, d in zip(sts, dj)]
                i0 = [umin(jnp.maximum(x, 0.0).astype(jnp.int32), H - 1)
                      for x in si]
                j0 = [umin(jnp.maximum(x, 0.0).astype(jnp.int32), W - 1)
                      for x in sj]
                i1 = [umin(x + 1, H - 1) for x in i0]
                j1 = [umin(x + 1, W - 1) for x in j0]
                dif = [x - v.astype(jnp.float32) for x, v in zip(si, i0)]
                djf = [x - v.astype(jnp.float32) for x, v in zip(sj, j0)]
                qw0 = [lax.shift_right_logical(a, 1) * W + b
                       for a, b in zip(i0, j0)]
                qw1 = [lax.shift_right_logical(a, 1) * W + b
                       for a, b in zip(i1, j0)]
                dj1 = [a - b for a, b in zip(j1, j0)]
                p0 = [lax.shift_left(jnp.bitwise_and(a, 1), 4) for a in i0]
                p1 = [lax.shift_left(jnp.bitwise_and(a, 1), 4) for a in i1]

                def tap(qw, psh):
                    w = plsc.load_gather(table_v, [qw])
                    return plsc.bitcast(lax.shift_left(w, psh), jnp.float32)

                f00 = [tap(q, p) for q, p in zip(qw0, p0)]
                f01 = [tap(q + d, p) for q, d, p in zip(qw0, dj1, p0)]
                f10 = [tap(q, p) for q, p in zip(qw1, p1)]
                f11 = [tap(q + d, p) for q, d, p in zip(qw1, dj1, p1)]
                f0 = [a + (b - a) * w for a, b, w in zip(f00, f01, djf)]
                f1 = [a + (b - a) * w for a, b, w in zip(f10, f11, djf)]
                for k, st in enumerate(sts):
                    out_v[r, pl.ds(st * L, L)] = (
                        f0[k] + (f1[k] - f0[k]) * dif[k])
        pltpu.sync_copy(out_v, out_hbm.at[b, f, pl.ds(row0, ROWS_PER_CHUNK)])
        return 0

    lax.fori_loop(0, NCHUNK, chunk_body, 0)


@functools.partial(
    pl.kernel,
    out_type=jax.ShapeDtypeStruct((B, F, H, W), jnp.float32),
    mesh=plsc.VectorSubcoreMesh(core_axis_name="c", subcore_axis_name="s"),
    compiler_params=pltpu.CompilerParams(needs_layout_passes=False),
    scratch_types=[
        pltpu.VMEM((TABLE_WORDS,), jnp.int32),
        pltpu.VMEM((ROWS_PER_CHUNK, WT, 2, WC), jnp.float32),
        pltpu.VMEM((ROWS_PER_CHUNK, W), jnp.float32),
        pltpu.VMEM((PACK_ROWS, W), jnp.float32),
    ],
)
def _warp_kernel(img_hbm, u_hbm, out_hbm, table_v, u_v, out_v, stage_v):
    _warp_body(img_hbm, u_hbm, out_hbm, table_v, u_v, out_v, stage_v)


def kernel(img, U):
    u = U.reshape(B, F, H, WT, WC, 2).transpose(0, 1, 2, 3, 5, 4)
    return _warp_kernel(img, u)


# speedup vs baseline: 11.2953x; 1.1877x over previous
import functools

import jax
import jax.numpy as jnp
from jax import lax
from jax.experimental import pallas as pl
from jax.experimental.pallas import tpu as pltpu
from jax.experimental.pallas import tpu_sc as plsc

B, F, H, W = 4, 8, 384, 384
NC, NS, L = 2, 16, 16
NW = NC * NS
WT, WC = W // 128, 128
ROWS_PER_CHUNK = 16
NCHUNK = H // ROWS_PER_CHUNK
NSTEP = W // L
PACK_ROWS = 32
NPCHUNK = H // PACK_ROWS
TABLE_WORDS = H * W // 2


def _warp_body(img_hbm, u_hbm, out_hbm, table_v, u_v, out_v, stage_v,
               sem_in, sem_out):
    c = lax.axis_index("c")
    s = lax.axis_index("s")
    wid = s * NC + c
    b = wid // F
    f = wid % F

    lane = lax.iota(jnp.int32, L)
    lane_f = lane.astype(jnp.float32)

    def pack_chunk(pk, _):
        pltpu.sync_copy(img_hbm.at[b, pl.ds(pk * PACK_ROWS, PACK_ROWS)], stage_v)

        @plsc.parallel_loop(0, PACK_ROWS // 2)
        def pack_row(rp):
            wrow = pk * (PACK_ROWS // 2) + rp
            for st in range(NSTEP):
                va = stage_v[2 * rp + 1, pl.ds(st * L, L)]
                vb = stage_v[2 * rp, pl.ds(st * L, L)]
                packed = plsc.pack(va, vb, format=plsc.PackFormat.INTERLEAVED)
                table_v[pl.ds(wrow * W + st * L, L)] = plsc.bitcast(
                    packed, jnp.int32)

        return 0

    lax.fori_loop(0, NPCHUNK, pack_chunk, 0)

    def u_copy(ck, slot):
        return pltpu.make_async_copy(
            u_hbm.at[b, f, pl.ds(ck * ROWS_PER_CHUNK, ROWS_PER_CHUNK)],
            u_v.at[slot], sem_in.at[slot])

    def out_copy(ck, slot):
        return pltpu.make_async_copy(
            out_v.at[slot],
            out_hbm.at[b, f, pl.ds(ck * ROWS_PER_CHUNK, ROWS_PER_CHUNK)],
            sem_out.at[slot])

    u_copy(0, 0).start()

    def chunk_pair(g, _):
        for slot in range(2):
            ck = g * 2 + slot
            row0 = ck * ROWS_PER_CHUNK
            u_copy(ck, slot).wait()

            @pl.when(ck + 1 < NCHUNK)
            def _():
                u_copy(ck + 1, 1 - slot).start()

            @pl.when(ck >= 2)
            def _():
                out_copy(ck - 2, slot).wait()

            _do_rows(row0, slot)
            out_copy(ck, slot).start()
        return 0

    def _do_rows(row0, slot):
        @plsc.parallel_loop(0, ROWS_PER_CHUNK)
        def row_body(r):
            row_f = lax.broadcast_in_dim(
                (row0 + r).astype(jnp.float32), (L,), ())

            def umin(x, k):
                return plsc.bitcast(
                    jnp.minimum(plsc.bitcast(x, jnp.uint32), jnp.uint32(k)),
                    jnp.int32)

            GROUP = 4
            for st0 in range(0, NSTEP, GROUP):
                sts = range(st0, st0 + GROUP)
                di = [u_v[slot, r, (st * L) // WC, 0, pl.ds((st * L) % WC, L)]
                      for st in sts]
                dj = [u_v[slot, r, (st * L) // WC, 1, pl.ds((st * L) % WC, L)]
                      for st in sts]
                si = [row_f - d for d in di]
                sj = [(jnp.float32(st * L) + lane_f) - d
                      for st, d in zip(sts, dj)]
                i0 = [umin(jnp.maximum(x, 0.0).astype(jnp.int32), H - 1)
                      for x in si]
                j0 = [umin(jnp.maximum(x, 0.0).astype(jnp.int32), W - 1)
                      for x in sj]
                i1 = [umin(x + 1, H - 1) for x in i0]
                j1 = [umin(x + 1, W - 1) for x in j0]
                dif = [x - v.astype(jnp.float32) for x, v in zip(si, i0)]
                djf = [x - v.astype(jnp.float32) for x, v in zip(sj, j0)]
                qw0 = [lax.shift_right_logical(a, 1) * W + b
                       for a, b in zip(i0, j0)]
                qw1 = [lax.shift_right_logical(a, 1) * W + b
                       for a, b in zip(i1, j0)]
                dj1 = [a - b for a, b in zip(j1, j0)]
                p0 = [lax.shift_left(jnp.bitwise_and(a, 1), 4) for a in i0]
                p1 = [lax.shift_left(jnp.bitwise_and(a, 1), 4) for a in i1]

                def tap(qw, psh):
                    w = plsc.load_gather(table_v, [qw])
                    return plsc.bitcast(lax.shift_left(w, psh), jnp.float32)

                f00 = [tap(q, p) for q, p in zip(qw0, p0)]
                f01 = [tap(q + d, p) for q, d, p in zip(qw0, dj1, p0)]
                f10 = [tap(q, p) for q, p in zip(qw1, p1)]
                f11 = [tap(q + d, p) for q, d, p in zip(qw1, dj1, p1)]
                f0 = [a + (b - a) * w for a, b, w in zip(f00, f01, djf)]
                f1 = [a + (b - a) * w for a, b, w in zip(f10, f11, djf)]
                for k, st in enumerate(sts):
                    out_v[slot, r, pl.ds(st * L, L)] = (
                        f0[k] + (f1[k] - f0[k]) * dif[k])

    lax.fori_loop(0, NCHUNK // 2, chunk_pair, 0)
    out_copy(NCHUNK - 2, 0).wait()
    out_copy(NCHUNK - 1, 1).wait()


@functools.partial(
    pl.kernel,
    out_type=jax.ShapeDtypeStruct((B, F, H, W), jnp.float32),
    mesh=plsc.VectorSubcoreMesh(core_axis_name="c", subcore_axis_name="s"),
    compiler_params=pltpu.CompilerParams(needs_layout_passes=False),
    scratch_types=[
        pltpu.VMEM((TABLE_WORDS,), jnp.int32),
        pltpu.VMEM((2, ROWS_PER_CHUNK, WT, 2, WC), jnp.float32),
        pltpu.VMEM((2, ROWS_PER_CHUNK, W), jnp.float32),
        pltpu.VMEM((PACK_ROWS, W), jnp.float32),
        pltpu.SemaphoreType.DMA((2,)),
        pltpu.SemaphoreType.DMA((2,)),
    ],
)
def _warp_kernel(img_hbm, u_hbm, out_hbm, table_v, u_v, out_v, stage_v,
                 sem_in, sem_out):
    _warp_body(img_hbm, u_hbm, out_hbm, table_v, u_v, out_v, stage_v,
               sem_in, sem_out)


def kernel(img, U):
    u = U.reshape(B, F, H, WT, WC, 2).transpose(0, 1, 2, 3, 5, 4)
    return _warp_kernel(img, u)


# speedup vs baseline: 13.3069x; 1.1781x over previous
import functools

import jax
import jax.numpy as jnp
from jax import lax
from jax.experimental import pallas as pl
from jax.experimental.pallas import tpu as pltpu
from jax.experimental.pallas import tpu_sc as plsc

B, F, H, W = 4, 8, 384, 384
NC, NS, L = 2, 16, 16
NW = NC * NS
WT, WC = W // 128, 128
ROWS_PER_CHUNK = 16
NCHUNK = H // ROWS_PER_CHUNK
NSTEP = W // L
PACK_ROWS = 32
NPCHUNK = H // PACK_ROWS
WP = W + 1
HP = H // 2 + 1
TABLE_WORDS = HP * WP


def _warp_body(img_hbm, u_hbm, out_hbm, table_v, u_v, out_v, stage_v,
               sem_in, sem_out):
    c = lax.axis_index("c")
    s = lax.axis_index("s")
    wid = s * NC + c
    b = wid // F
    f = wid % F

    lane = lax.iota(jnp.int32, L)
    lane_f = lane.astype(jnp.float32)

    def pack_chunk(pk, _):
        pltpu.sync_copy(img_hbm.at[b, pl.ds(pk * PACK_ROWS, PACK_ROWS)], stage_v)

        @plsc.parallel_loop(0, PACK_ROWS // 2)
        def pack_row(rp):
            wrow = pk * (PACK_ROWS // 2) + rp
            for st in range(NSTEP):
                va = stage_v[2 * rp + 1, pl.ds(st * L, L)]
                vb = stage_v[2 * rp, pl.ds(st * L, L)]
                packed = plsc.pack(va, vb, format=plsc.PackFormat.INTERLEAVED)
                table_v[pl.ds(wrow * WP + st * L, L)] = plsc.bitcast(
                    packed, jnp.int32)
            last = lax.broadcast_in_dim(wrow * WP + (W - 1), (L,), ())
            wlast = plsc.load_gather(table_v, [last])
            plsc.store_scatter(table_v, [last + 1], wlast)

        return 0

    lax.fori_loop(0, NPCHUNK, pack_chunk, 0)

    pltpu.sync_copy(img_hbm.at[b, pl.ds(H - 16, 16)],
                    stage_v.at[pl.ds(0, 16)])
    for st in range(NSTEP):
        vlast = stage_v[15, pl.ds(st * L, L)]
        packed = plsc.pack(vlast, vlast, format=plsc.PackFormat.INTERLEAVED)
        table_v[pl.ds((HP - 1) * WP + st * L, L)] = plsc.bitcast(
            packed, jnp.int32)
    lastw = lax.broadcast_in_dim(jnp.int32((HP - 1) * WP + (W - 1)), (L,), ())
    plsc.store_scatter(table_v, [lastw + 1], plsc.load_gather(table_v, [lastw]))

    def u_copy(ck, slot):
        return pltpu.make_async_copy(
            u_hbm.at[b, f, pl.ds(ck * ROWS_PER_CHUNK, ROWS_PER_CHUNK)],
            u_v.at[slot], sem_in.at[slot])

    def out_copy(ck, slot):
        return pltpu.make_async_copy(
            out_v.at[slot],
            out_hbm.at[b, f, pl.ds(ck * ROWS_PER_CHUNK, ROWS_PER_CHUNK)],
            sem_out.at[slot])

    u_copy(0, 0).start()

    def chunk_pair(g, _):
        for slot in range(2):
            ck = g * 2 + slot
            row0 = ck * ROWS_PER_CHUNK
            u_copy(ck, slot).wait()

            @pl.when(ck + 1 < NCHUNK)
            def _():
                u_copy(ck + 1, 1 - slot).start()

            @pl.when(ck >= 2)
            def _():
                out_copy(ck - 2, slot).wait()

            _do_rows(row0, slot)
            out_copy(ck, slot).start()
        return 0

    def _do_rows(row0, slot):
        @plsc.parallel_loop(0, ROWS_PER_CHUNK)
        def row_body(r):
            row_f = lax.broadcast_in_dim(
                (row0 + r).astype(jnp.float32), (L,), ())

            def umin(x, k):
                return plsc.bitcast(
                    jnp.minimum(plsc.bitcast(x, jnp.uint32), jnp.uint32(k)),
                    jnp.int32)

            GROUP = 4
            for st0 in range(0, NSTEP, GROUP):
                sts = range(st0, st0 + GROUP)
                di = [u_v[slot, r, (st * L) // WC, 0, pl.ds((st * L) % WC, L)]
                      for st in sts]
                dj = [u_v[slot, r, (st * L) // WC, 1, pl.ds((st * L) % WC, L)]
                      for st in sts]
                si = [row_f - d for d in di]
                sj = [(lane_f - d) + jnp.float32(st * L) if st else lane_f - d
                      for st, d in zip(sts, dj)]
                i0 = [umin(jnp.maximum(x, 0.0).astype(jnp.int32), H - 1)
                      for x in si]
                j0 = [umin(jnp.maximum(x, 0.0).astype(jnp.int32), W - 1)
                      for x in sj]
                i1 = [x + 1 for x in i0]
                dif = [x - v.astype(jnp.float32) for x, v in zip(si, i0)]
                djf = [x - v.astype(jnp.float32) for x, v in zip(sj, j0)]
                qw0 = [lax.shift_right_logical(a, 1) * WP + b
                       for a, b in zip(i0, j0)]
                qw1 = [lax.shift_right_logical(a, 1) * WP + b
                       for a, b in zip(i1, j0)]
                p0 = [lax.shift_left(jnp.bitwise_and(a, 1), 4) for a in i0]
                p1 = [lax.shift_left(jnp.bitwise_and(a, 1), 4) for a in i1]

                def tap(qw, psh):
                    w = plsc.load_gather(table_v, [qw])
                    return plsc.bitcast(lax.shift_left(w, psh), jnp.float32)

                f00 = [tap(q, p) for q, p in zip(qw0, p0)]
                f01 = [tap(q + 1, p) for q, p in zip(qw0, p0)]
                f10 = [tap(q, p) for q, p in zip(qw1, p1)]
                f11 = [tap(q + 1, p) for q, p in zip(qw1, p1)]
                f0 = [a + (b - a) * w for a, b, w in zip(f00, f01, djf)]
                f1 = [a + (b - a) * w for a, b, w in zip(f10, f11, djf)]
                for k, st in enumerate(sts):
                    out_v[slot, r, pl.ds(st * L, L)] = (
                        f0[k] + (f1[k] - f0[k]) * dif[k])

    lax.fori_loop(0, NCHUNK // 2, chunk_pair, 0)
    out_copy(NCHUNK - 2, 0).wait()
    out_copy(NCHUNK - 1, 1).wait()


@functools.partial(
    pl.kernel,
    out_type=jax.ShapeDtypeStruct((B, F, H, W), jnp.float32),
    mesh=plsc.VectorSubcoreMesh(core_axis_name="c", subcore_axis_name="s"),
    compiler_params=pltpu.CompilerParams(needs_layout_passes=False),
    scratch_types=[
        pltpu.VMEM((TABLE_WORDS,), jnp.int32),
        pltpu.VMEM((2, ROWS_PER_CHUNK, WT, 2, WC), jnp.float32),
        pltpu.VMEM((2, ROWS_PER_CHUNK, W), jnp.float32),
        pltpu.VMEM((PACK_ROWS, W), jnp.float32),
        pltpu.SemaphoreType.DMA((2,)),
        pltpu.SemaphoreType.DMA((2,)),
    ],
)
def _warp_kernel(img_hbm, u_hbm, out_hbm, table_v, u_v, out_v, stage_v,
                 sem_in, sem_out):
    _warp_body(img_hbm, u_hbm, out_hbm, table_v, u_v, out_v, stage_v,
               sem_in, sem_out)


def kernel(img, U):
    u = U.reshape(B, F, H, WT, WC, 2).transpose(0, 1, 2, 3, 5, 4)
    return _warp_kernel(img, u)


# speedup vs baseline: 13.3204x; 1.0010x over previous
import functools

import jax
import jax.numpy as jnp
from jax import lax
from jax.experimental import pallas as pl
from jax.experimental.pallas import tpu as pltpu
from jax.experimental.pallas import tpu_sc as plsc

B, F, H, W = 4, 8, 384, 384
NC, NS, L = 2, 16, 16
NW = NC * NS
WT, WC = W // 128, 128
ROWS_PER_CHUNK = 16
NCHUNK = H // ROWS_PER_CHUNK
NSTEP = W // L
PACK_ROWS = 16
NPCHUNK = H // PACK_ROWS
WP = W + 1
HP = H // 2 + 1
TABLE_WORDS = HP * WP


def _warp_body(img_hbm, u_hbm, out_hbm, table_v, u_v, out_v, stage_v,
               sem_in, sem_out, sem_pack):
    c = lax.axis_index("c")
    s = lax.axis_index("s")
    wid = s * NC + c
    b = wid // F
    f = wid % F

    lane = lax.iota(jnp.int32, L)
    lane_f = lane.astype(jnp.float32)

    def img_copy(pk, slot):
        return pltpu.make_async_copy(
            img_hbm.at[b, pl.ds(pk * PACK_ROWS, PACK_ROWS)],
            stage_v.at[slot], sem_pack.at[slot])

    def u_copy(ck, slot):
        return pltpu.make_async_copy(
            u_hbm.at[b, f, pl.ds(ck * ROWS_PER_CHUNK, ROWS_PER_CHUNK)],
            u_v.at[slot], sem_in.at[slot])

    def out_copy(ck, slot):
        return pltpu.make_async_copy(
            out_v.at[slot],
            out_hbm.at[b, f, pl.ds(ck * ROWS_PER_CHUNK, ROWS_PER_CHUNK)],
            sem_out.at[slot])

    img_copy(0, 0).start()
    u_copy(0, 0).start()

    def pack_pair(g, _):
        for slot in range(2):
            pk = g * 2 + slot
            img_copy(pk, slot).wait()

            @pl.when(pk + 1 < NPCHUNK)
            def _():
                img_copy(pk + 1, 1 - slot).start()

            @plsc.parallel_loop(0, PACK_ROWS // 2)
            def pack_row(rp):
                wrow = pk * (PACK_ROWS // 2) + rp
                for st in range(NSTEP):
                    va = stage_v[slot, 2 * rp + 1, pl.ds(st * L, L)]
                    vb = stage_v[slot, 2 * rp, pl.ds(st * L, L)]
                    packed = plsc.pack(
                        va, vb, format=plsc.PackFormat.INTERLEAVED)
                    table_v[pl.ds(wrow * WP + st * L, L)] = plsc.bitcast(
                        packed, jnp.int32)
                last = lax.broadcast_in_dim(wrow * WP + (W - 1), (L,), ())
                wlast = plsc.load_gather(table_v, [last])
                plsc.store_scatter(table_v, [last + 1], wlast)

        return 0

    lax.fori_loop(0, NPCHUNK // 2, pack_pair, 0)

    for st in range(NSTEP):
        vlast = stage_v[1, PACK_ROWS - 1, pl.ds(st * L, L)]
        packed = plsc.pack(vlast, vlast, format=plsc.PackFormat.INTERLEAVED)
        table_v[pl.ds((HP - 1) * WP + st * L, L)] = plsc.bitcast(
            packed, jnp.int32)
    lastw = lax.broadcast_in_dim(jnp.int32((HP - 1) * WP + (W - 1)), (L,), ())
    plsc.store_scatter(table_v, [lastw + 1], plsc.load_gather(table_v, [lastw]))

    def chunk_pair(g, _):
        for slot in range(2):
            ck = g * 2 + slot
            row0 = ck * ROWS_PER_CHUNK
            u_copy(ck, slot).wait()

            @pl.when(ck + 1 < NCHUNK)
            def _():
                u_copy(ck + 1, 1 - slot).start()

            @pl.when(ck >= 2)
            def _():
                out_copy(ck - 2, slot).wait()

            _do_rows(row0, slot)
            out_copy(ck, slot).start()
        return 0

    def _do_rows(row0, slot):
        @plsc.parallel_loop(0, ROWS_PER_CHUNK)
        def row_body(r):
            row_f = lax.broadcast_in_dim(
                (row0 + r).astype(jnp.float32), (L,), ())

            def umin(x, k):
                return plsc.bitcast(
                    jnp.minimum(plsc.bitcast(x, jnp.uint32), jnp.uint32(k)),
                    jnp.int32)

            GROUP = 4
            for st0 in range(0, NSTEP, GROUP):
                sts = range(st0, st0 + GROUP)
                di = [u_v[slot, r, (st * L) // WC, 0, pl.ds((st * L) % WC, L)]
                      for st in sts]
                dj = [u_v[slot, r, (st * L) // WC, 1, pl.ds((st * L) % WC, L)]
                      for st in sts]
                si = [row_f - d for d in di]
                sj = [(lane_f - d) + jnp.float32(st * L) if st else lane_f - d
                      for st, d in zip(sts, dj)]
                i0 = [umin(jnp.maximum(x, 0.0).astype(jnp.int32), H - 1)
                      for x in si]
                j0 = [umin(jnp.maximum(x, 0.0).astype(jnp.int32), W - 1)
                      for x in sj]
                i1 = [x + 1 for x in i0]
                dif = [x - v.astype(jnp.float32) for x, v in zip(si, i0)]
                djf = [x - v.astype(jnp.float32) for x, v in zip(sj, j0)]
                qw0 = [lax.shift_right_logical(a, 1) * WP + b
                       for a, b in zip(i0, j0)]
                qw1 = [lax.shift_right_logical(a, 1) * WP + b
                       for a, b in zip(i1, j0)]
                p0 = [lax.shift_left(jnp.bitwise_and(a, 1), 4) for a in i0]
                p1 = [lax.shift_left(jnp.bitwise_and(a, 1), 4) for a in i1]

                def tap(qw, psh):
                    w = plsc.load_gather(table_v, [qw])
                    return plsc.bitcast(lax.shift_left(w, psh), jnp.float32)

                f00 = [tap(q, p) for q, p in zip(qw0, p0)]
                f01 = [tap(q + 1, p) for q, p in zip(qw0, p0)]
                f10 = [tap(q, p) for q, p in zip(qw1, p1)]
                f11 = [tap(q + 1, p) for q, p in zip(qw1, p1)]
                f0 = [a + (b - a) * w for a, b, w in zip(f00, f01, djf)]
                f1 = [a + (b - a) * w for a, b, w in zip(f10, f11, djf)]
                for k, st in enumerate(sts):
                    out_v[slot, r, pl.ds(st * L, L)] = (
                        f0[k] + (f1[k] - f0[k]) * dif[k])

    lax.fori_loop(0, NCHUNK // 2, chunk_pair, 0)
    out_copy(NCHUNK - 2, 0).wait()
    out_copy(NCHUNK - 1, 1).wait()


@functools.partial(
    pl.kernel,
    out_type=jax.ShapeDtypeStruct((B, F, H, W), jnp.float32),
    mesh=plsc.VectorSubcoreMesh(core_axis_name="c", subcore_axis_name="s"),
    compiler_params=pltpu.CompilerParams(needs_layout_passes=False),
    scratch_types=[
        pltpu.VMEM((TABLE_WORDS,), jnp.int32),
        pltpu.VMEM((2, ROWS_PER_CHUNK, WT, 2, WC), jnp.float32),
        pltpu.VMEM((2, ROWS_PER_CHUNK, W), jnp.float32),
        pltpu.VMEM((2, PACK_ROWS, W), jnp.float32),
        pltpu.SemaphoreType.DMA((2,)),
        pltpu.SemaphoreType.DMA((2,)),
        pltpu.SemaphoreType.DMA((2,)),
    ],
)
def _warp_kernel(img_hbm, u_hbm, out_hbm, table_v, u_v, out_v, stage_v,
                 sem_in, sem_out, sem_pack):
    _warp_body(img_hbm, u_hbm, out_hbm, table_v, u_v, out_v, stage_v,
               sem_in, sem_out, sem_pack)


def kernel(img, U):
    u = U.reshape(B, F, H, WT, WC, 2).transpose(0, 1, 2, 3, 5, 4)
    return _warp_kernel(img, u)


# speedup vs baseline: 13.6167x; 1.0222x over previous
import functools

import jax
import jax.numpy as jnp
from jax import lax
from jax.experimental import pallas as pl
from jax.experimental.pallas import tpu as pltpu
from jax.experimental.pallas import tpu_sc as plsc

B, F, H, W = 4, 8, 384, 384
NC, NS, L = 2, 16, 16
NW = NC * NS
WT, WC = W // 128, 128
ROWS_PER_CHUNK = 16
NCHUNK = H // ROWS_PER_CHUNK
NSTEP = W // L
PACK_ROWS = 16
NPCHUNK = H // PACK_ROWS
WP = W + 1
HP = H // 2 + 1
TABLE_WORDS = HP * WP


def _warp_body(img_hbm, u_hbm, out_hbm, table_v, u_v, out_v, stage_v,
               sem_in, sem_out, sem_pack):
    c = lax.axis_index("c")
    s = lax.axis_index("s")
    wid = s * NC + c
    b = wid // F
    f = wid % F

    lane = lax.iota(jnp.int32, L)
    lane_f = lane.astype(jnp.float32)

    def img_copy(pk, slot):
        return pltpu.make_async_copy(
            img_hbm.at[b, pl.ds(pk * PACK_ROWS, PACK_ROWS)],
            stage_v.at[slot], sem_pack.at[slot])

    def u_copy(ck, slot):
        return pltpu.make_async_copy(
            u_hbm.at[b, f, pl.ds(ck * ROWS_PER_CHUNK, ROWS_PER_CHUNK)],
            u_v.at[slot], sem_in.at[slot])

    def out_copy(ck, slot):
        return pltpu.make_async_copy(
            out_v.at[slot],
            out_hbm.at[b, f, pl.ds(ck * ROWS_PER_CHUNK, ROWS_PER_CHUNK)],
            sem_out.at[slot])

    img_copy(0, 0).start()
    u_copy(0, 0).start()

    def pack_pair(g, _):
        for slot in range(2):
            pk = g * 2 + slot
            img_copy(pk, slot).wait()

            @pl.when(pk + 1 < NPCHUNK)
            def _():
                img_copy(pk + 1, 1 - slot).start()

            @plsc.parallel_loop(0, PACK_ROWS // 2)
            def pack_row(rp):
                wrow = pk * (PACK_ROWS // 2) + rp
                for st in range(NSTEP):
                    va = stage_v[slot, 2 * rp + 1, pl.ds(st * L, L)]
                    vb = stage_v[slot, 2 * rp, pl.ds(st * L, L)]
                    packed = plsc.pack(
                        va, vb, format=plsc.PackFormat.INTERLEAVED)
                    table_v[pl.ds(wrow * WP + st * L, L)] = plsc.bitcast(
                        packed, jnp.int32)
                last = lax.broadcast_in_dim(wrow * WP + (W - 1), (L,), ())
                wlast = plsc.load_gather(table_v, [last])
                plsc.store_scatter(table_v, [last + 1], wlast)

        return 0

    lax.fori_loop(0, NPCHUNK // 2, pack_pair, 0)

    for st in range(NSTEP):
        vlast = stage_v[1, PACK_ROWS - 1, pl.ds(st * L, L)]
        packed = plsc.pack(vlast, vlast, format=plsc.PackFormat.INTERLEAVED)
        table_v[pl.ds((HP - 1) * WP + st * L, L)] = plsc.bitcast(
            packed, jnp.int32)
    lastw = lax.broadcast_in_dim(jnp.int32((HP - 1) * WP + (W - 1)), (L,), ())
    plsc.store_scatter(table_v, [lastw + 1], plsc.load_gather(table_v, [lastw]))

    def chunk_pair(g, _):
        for slot in range(2):
            ck = g * 2 + slot
            row0 = ck * ROWS_PER_CHUNK
            u_copy(ck, slot).wait()

            @pl.when(ck + 1 < NCHUNK)
            def _():
                u_copy(ck + 1, 1 - slot).start()

            @pl.when(ck >= 2)
            def _():
                out_copy(ck - 2, slot).wait()

            _do_rows(row0, slot)
            out_copy(ck, slot).start()
        return 0

    def _do_rows(row0, slot):
        @plsc.parallel_loop(0, ROWS_PER_CHUNK)
        def row_body(r):
            row_f = lax.broadcast_in_dim(
                (row0 + r).astype(jnp.float32), (L,), ())

            def umin(x, k):
                return plsc.bitcast(
                    jnp.minimum(plsc.bitcast(x, jnp.uint32), jnp.uint32(k)),
                    jnp.int32)

            GROUP = 6
            for st0 in range(0, NSTEP, GROUP):
                sts = range(st0, st0 + GROUP)
                di = [u_v[slot, r, (st * L) // WC, 0, pl.ds((st * L) % WC, L)]
                      for st in sts]
                dj = [u_v[slot, r, (st * L) // WC, 1, pl.ds((st * L) % WC, L)]
                      for st in sts]
                si = [row_f - d for d in di]
                sj = [(lane_f - d) + jnp.float32(st * L) if st else lane_f - d
                      for st, d in zip(sts, dj)]
                i0 = [umin(jnp.maximum(x, 0.0).astype(jnp.int32), H - 1)
                      for x in si]
                j0 = [umin(jnp.maximum(x, 0.0).astype(jnp.int32), W - 1)
                      for x in sj]
                i1 = [x + 1 for x in i0]
                dif = [x - v.astype(jnp.float32) for x, v in zip(si, i0)]
                djf = [x - v.astype(jnp.float32) for x, v in zip(sj, j0)]
                qw0 = [lax.shift_right_logical(a, 1) * WP + b
                       for a, b in zip(i0, j0)]
                qw1 = [lax.shift_right_logical(a, 1) * WP + b
                       for a, b in zip(i1, j0)]
                p0 = [lax.shift_left(jnp.bitwise_and(a, 1), 4) for a in i0]
                p1 = [lax.shift_left(jnp.bitwise_and(a, 1), 4) for a in i1]

                def tap(qw, psh):
                    w = plsc.load_gather(table_v, [qw])
                    return plsc.bitcast(lax.shift_left(w, psh), jnp.float32)

                f00 = [tap(q, p) for q, p in zip(qw0, p0)]
                f01 = [tap(q + 1, p) for q, p in zip(qw0, p0)]
                f10 = [tap(q, p) for q, p in zip(qw1, p1)]
                f11 = [tap(q + 1, p) for q, p in zip(qw1, p1)]
                f0 = [a + (b - a) * w for a, b, w in zip(f00, f01, djf)]
                f1 = [a + (b - a) * w for a, b, w in zip(f10, f11, djf)]
                for k, st in enumerate(sts):
                    out_v[slot, r, pl.ds(st * L, L)] = (
                        f0[k] + (f1[k] - f0[k]) * dif[k])

    lax.fori_loop(0, NCHUNK // 2, chunk_pair, 0)
    out_copy(NCHUNK - 2, 0).wait()
    out_copy(NCHUNK - 1, 1).wait()


@functools.partial(
    pl.kernel,
    out_type=jax.ShapeDtypeStruct((B, F, H, W), jnp.float32),
    mesh=plsc.VectorSubcoreMesh(core_axis_name="c", subcore_axis_name="s"),
    compiler_params=pltpu.CompilerParams(needs_layout_passes=False),
    scratch_types=[
        pltpu.VMEM((TABLE_WORDS,), jnp.int32),
        pltpu.VMEM((2, ROWS_PER_CHUNK, WT, 2, WC), jnp.float32),
        pltpu.VMEM((2, ROWS_PER_CHUNK, W), jnp.float32),
        pltpu.VMEM((2, PACK_ROWS, W), jnp.float32),
        pltpu.SemaphoreType.DMA((2,)),
        pltpu.SemaphoreType.DMA((2,)),
        pltpu.SemaphoreType.DMA((2,)),
    ],
)
def _warp_kernel(img_hbm, u_hbm, out_hbm, table_v, u_v, out_v, stage_v,
                 sem_in, sem_out, sem_pack):
    _warp_body(img_hbm, u_hbm, out_hbm, table_v, u_v, out_v, stage_v,
               sem_in, sem_out, sem_pack)


def kernel(img, U):
    u = U.reshape(B, F, H, WT, WC, 2).transpose(0, 1, 2, 3, 5, 4)
    return _warp_kernel(img, u)
